# Initial kernel scaffold; baseline (speedup 1.0000x reference)
import functools

import jax
import jax.numpy as jnp
from jax import lax
from jax.experimental import pallas as pl
from jax.experimental.pallas import tpu as pltpu
from jax.experimental.pallas import tpu_sc as plsc

N_NODES = 100000
F_X = 128
N_GRAPHS = 64
F_OUT = 128

NC = 2
NS = 16
NW = NC * NS

CHUNK = 128
N_CHUNKS = N_NODES // CHUNK
N_MAIN = N_CHUNKS * CHUNK
TAIL = N_NODES - N_MAIN


def _sc_segment_partials(x, batch):
    mesh = plsc.VectorSubcoreMesh(
        core_axis_name="c", subcore_axis_name="s", num_cores=NC, num_subcores=NS
    )

    @functools.partial(
        pl.kernel,
        out_type=jax.ShapeDtypeStruct((NW, N_GRAPHS, F_X), jnp.float32),
        mesh=mesh,
        scratch_types=[
            pltpu.VMEM((CHUNK, F_X), jnp.float32),
            pltpu.VMEM((CHUNK,), jnp.int32),
            pltpu.VMEM((N_GRAPHS, F_X), jnp.float32),
            pltpu.VMEM((TAIL, F_X), jnp.float32),
            pltpu.VMEM((TAIL,), jnp.int32),
        ],
    )
    def sc_kernel(x_hbm, b_hbm, out_hbm, rows_v, idx_v, acc_v, trow_v, tidx_v):
        wid = lax.axis_index("s") * NC + lax.axis_index("c")

        zeros = jnp.zeros((16,), jnp.float32)

        def zero_row(g, carry):
            for f in range(F_X // 16):
                acc_v[g, pl.ds(16 * f, 16)] = zeros
            return carry

        lax.fori_loop(0, N_GRAPHS, zero_row, 0)

        def body(i, carry):
            base = (wid + i * NW) * CHUNK
            pltpu.sync_copy(b_hbm.at[pl.ds(base, CHUNK)], idx_v)
            pltpu.sync_copy(x_hbm.at[pl.ds(base, CHUNK)], rows_v)
            pltpu.sync_copy(rows_v, acc_v.at[idx_v], add=True)
            return carry

        n_mine = (N_CHUNKS - 1 - wid) // NW + 1
        lax.fori_loop(0, n_mine, body, 0)

        @pl.when(wid == 0)
        def _tail():
            pltpu.sync_copy(b_hbm.at[pl.ds(N_MAIN, TAIL)], tidx_v)
            pltpu.sync_copy(x_hbm.at[pl.ds(N_MAIN, TAIL)], trow_v)
            pltpu.sync_copy(trow_v, acc_v.at[tidx_v], add=True)

        pltpu.sync_copy(acc_v, out_hbm.at[wid])

    return sc_kernel(x, batch)


def _tc_head(partials, u, W, b2d):
    def tc_kernel(p_ref, u_ref, w_ref, b_ref, o_ref):
        x_agg = jnp.sum(p_ref[...], axis=0)
        out = jnp.dot(x_agg, w_ref[:F_X, :], preferred_element_type=jnp.float32)
        out = out + jnp.dot(u_ref[...], w_ref[F_X:, :], preferred_element_type=jnp.float32)
        o_ref[...] = out + b_ref[...]

    return pl.pallas_call(
        tc_kernel,
        out_shape=jax.ShapeDtypeStruct((N_GRAPHS, F_OUT), jnp.float32),
    )(partials, u, W, b2d)


def kernel(x, edge_index, e, u, batch, W, b):
    del edge_index, e
    batch32 = batch.astype(jnp.int32)
    partials = _sc_segment_partials(x, batch32)
    return _tc_head(partials, u, W, b.reshape(1, F_OUT).astype(jnp.float32))


# speedup vs baseline: 4.2397x; 4.2397x over previous
import functools

import jax
import jax.numpy as jnp
from jax import lax
from jax.experimental import pallas as pl
from jax.experimental.pallas import tpu as pltpu
from jax.experimental.pallas import tpu_sc as plsc

N_NODES = 100000
F_X = 128
N_GRAPHS = 64
F_OUT = 128

NC = 2
NS = 16
NW = NC * NS

CHUNK = 128
N_CHUNKS = N_NODES // CHUNK
N_MAIN = N_CHUNKS * CHUNK
TAIL = N_NODES - N_MAIN


def _sc_segment_partials(x, batch):
    mesh = plsc.VectorSubcoreMesh(
        core_axis_name="c", subcore_axis_name="s", num_cores=NC, num_subcores=NS
    )
    zrows = N_GRAPHS // NS

    @functools.partial(
        pl.kernel,
        out_type=jax.ShapeDtypeStruct((NC, N_GRAPHS, F_X), jnp.float32),
        mesh=mesh,
        scratch_types=[
            pltpu.VMEM((CHUNK, F_X), jnp.float32),
            pltpu.VMEM((CHUNK,), jnp.int32),
            pltpu.VMEM((zrows, F_X), jnp.float32),
            pltpu.VMEM((TAIL, F_X), jnp.float32),
            pltpu.VMEM((TAIL,), jnp.int32),
            pltpu.VMEM_SHARED((N_GRAPHS, F_X), jnp.float32),
        ],
    )
    def sc_kernel(x_hbm, b_hbm, out_hbm, rows_v, idx_v, zbuf_v, trow_v, tidx_v, acc_sh):
        cid = lax.axis_index("c")
        sid = lax.axis_index("s")
        wid = sid * NC + cid

        zeros = jnp.zeros((16,), jnp.float32)
        for r in range(zrows):
            for f in range(F_X // 16):
                zbuf_v[r, pl.ds(16 * f, 16)] = zeros
        pltpu.sync_copy(zbuf_v, acc_sh.at[pl.ds(sid * zrows, zrows)])
        plsc.subcore_barrier()

        def body(i, carry):
            base = (wid + i * NW) * CHUNK
            pltpu.sync_copy(b_hbm.at[pl.ds(base, CHUNK)], idx_v)
            pltpu.sync_copy(x_hbm.at[pl.ds(base, CHUNK)], rows_v)
            pltpu.sync_copy(rows_v, acc_sh.at[idx_v], add=True)
            return carry

        n_mine = (N_CHUNKS - 1 - wid) // NW + 1
        lax.fori_loop(0, n_mine, body, 0)

        @pl.when(wid == 0)
        def _tail():
            pltpu.sync_copy(b_hbm.at[pl.ds(N_MAIN, TAIL)], tidx_v)
            pltpu.sync_copy(x_hbm.at[pl.ds(N_MAIN, TAIL)], trow_v)
            pltpu.sync_copy(trow_v, acc_sh.at[tidx_v], add=True)

        plsc.subcore_barrier()

        @pl.when(sid == 0)
        def _writeback():
            pltpu.sync_copy(acc_sh, out_hbm.at[cid])

    return sc_kernel(x, batch)


def _tc_head(partials, u, W, b2d):
    def tc_kernel(p_ref, u_ref, w_ref, b_ref, o_ref):
        x_agg = p_ref[0] + p_ref[1]
        out = jnp.dot(x_agg, w_ref[:F_X, :], preferred_element_type=jnp.float32)
        out = out + jnp.dot(u_ref[...], w_ref[F_X:, :], preferred_element_type=jnp.float32)
        o_ref[...] = out + b_ref[...]

    return pl.pallas_call(
        tc_kernel,
        out_shape=jax.ShapeDtypeStruct((N_GRAPHS, F_OUT), jnp.float32),
    )(partials, u, W, b2d)


def kernel(x, edge_index, e, u, batch, W, b):
    del edge_index, e
    batch32 = batch.astype(jnp.int32)
    partials = _sc_segment_partials(x, batch32)
    return _tc_head(partials, u, W, b.reshape(1, F_OUT).astype(jnp.float32))


# speedup vs baseline: 6.9388x; 1.6366x over previous
import functools

import jax
import jax.numpy as jnp
from jax import lax
from jax.experimental import pallas as pl
from jax.experimental.pallas import tpu as pltpu
from jax.experimental.pallas import tpu_sc as plsc

N_NODES = 100000
F_X = 128
N_GRAPHS = 64
F_OUT = 128

NC = 2
NS = 16
NW = NC * NS

CHUNK = 128
N_CHUNKS = N_NODES // CHUNK
N_MAIN = N_CHUNKS * CHUNK
TAIL = N_NODES - N_MAIN


STEPS = -(-N_CHUNKS // NW)
N_FULL_W = N_CHUNKS - (STEPS - 1) * NW
NBUF = 4


def _sc_segment_partials(x, idx_by_worker):
    mesh = plsc.VectorSubcoreMesh(
        core_axis_name="c", subcore_axis_name="s", num_cores=NC, num_subcores=NS
    )
    zrows = N_GRAPHS // NS

    @functools.partial(
        pl.kernel,
        out_type=jax.ShapeDtypeStruct((NC, N_GRAPHS, F_X), jnp.float32),
        mesh=mesh,
        scratch_types=[
            pltpu.VMEM((NBUF, CHUNK, F_X), jnp.float32),
            pltpu.VMEM((STEPS, CHUNK), jnp.int32),
            pltpu.VMEM((zrows, F_X), jnp.float32),
            pltpu.VMEM((TAIL, F_X), jnp.float32),
            pltpu.VMEM((TAIL,), jnp.int32),
            pltpu.VMEM_SHARED((N_GRAPHS, F_X), jnp.float32),
            [pltpu.SemaphoreType.DMA] * NBUF,
            [pltpu.SemaphoreType.DMA] * NBUF,
        ],
    )
    def sc_kernel(x_hbm, i_hbm, out_hbm, rows_v, idx_v, zbuf_v, trow_v, tidx_v,
                  acc_sh, lsem, ssem):
        cid = lax.axis_index("c")
        sid = lax.axis_index("s")
        wid = sid * NC + cid
        valid_last = wid < N_FULL_W

        def maybe(i, fn):
            if i == STEPS - 1:
                pl.when(valid_last)(fn)
            else:
                fn()

        def issue_load(i):
            base = (wid + i * NW) * CHUNK
            pltpu.async_copy(x_hbm.at[pl.ds(base, CHUNK)], rows_v.at[i % NBUF],
                             lsem[i % NBUF])

        def wait_load(i):
            base = (wid + i * NW) * CHUNK
            pltpu.make_async_copy(x_hbm.at[pl.ds(base, CHUNK)],
                                  rows_v.at[i % NBUF], lsem[i % NBUF]).wait()

        def issue_scatter(i):
            pltpu.async_copy(rows_v.at[i % NBUF], acc_sh.at[idx_v.at[i]],
                             ssem[i % NBUF], add=True)

        def wait_scatter(i):
            pltpu.make_async_copy(rows_v.at[i % NBUF], acc_sh.at[idx_v.at[i]],
                                  ssem[i % NBUF]).wait()

        for i in range(NBUF - 1):
            issue_load(i)
        pltpu.sync_copy(i_hbm.at[wid], idx_v)

        zeros = jnp.zeros((16,), jnp.float32)
        for r in range(zrows):
            for f in range(F_X // 16):
                zbuf_v[r, pl.ds(16 * f, 16)] = zeros
        pltpu.sync_copy(zbuf_v, acc_sh.at[pl.ds(sid * zrows, zrows)])
        plsc.subcore_barrier()

        for i in range(STEPS):
            def step(i=i):
                wait_load(i)
                issue_scatter(i)
            maybe(i, step)
            f = i + NBUF - 1
            if f < STEPS:
                def prefetch(f=f):
                    if f >= NBUF:
                        wait_scatter(f - NBUF)
                    issue_load(f)
                maybe(f, prefetch)

        for i in range(STEPS - NBUF, STEPS):
            maybe(i, lambda i=i: wait_scatter(i))

        @pl.when(wid == 0)
        def _tail():
            pltpu.sync_copy(i_hbm.at[N_CHUNKS % NW, STEPS - 1, pl.ds(0, TAIL)],
                            tidx_v)
            pltpu.sync_copy(x_hbm.at[pl.ds(N_MAIN, TAIL)], trow_v)
            pltpu.sync_copy(trow_v, acc_sh.at[tidx_v], add=True)

        plsc.subcore_barrier()

        @pl.when(sid == 0)
        def _writeback():
            pltpu.sync_copy(acc_sh, out_hbm.at[cid])

    return sc_kernel(x, idx_by_worker)


def _tc_head(partials, u, W, b2d):
    def tc_kernel(p_ref, u_ref, w_ref, b_ref, o_ref):
        x_agg = p_ref[0] + p_ref[1]
        out = jnp.dot(x_agg, w_ref[:F_X, :], preferred_element_type=jnp.float32)
        out = out + jnp.dot(u_ref[...], w_ref[F_X:, :], preferred_element_type=jnp.float32)
        o_ref[...] = out + b_ref[...]

    return pl.pallas_call(
        tc_kernel,
        out_shape=jax.ShapeDtypeStruct((N_GRAPHS, F_OUT), jnp.float32),
    )(partials, u, W, b2d)


def kernel(x, edge_index, e, u, batch, W, b):
    del edge_index, e
    batch32 = batch.astype(jnp.int32)
    pad = STEPS * NW * CHUNK - N_NODES
    idx_by_worker = (
        jnp.pad(batch32, (0, pad))
        .reshape(STEPS, NW, CHUNK)
        .transpose(1, 0, 2)
    )
    partials = _sc_segment_partials(x, idx_by_worker)
    return _tc_head(partials, u, W, b.reshape(1, F_OUT).astype(jnp.float32))
